# Initial kernel scaffold; baseline (speedup 1.0000x reference)
import jax
import jax.numpy as jnp
from jax.experimental import pallas as pl


def kernel(nnz, row_ptr_s, col_idx_s, edge_val_csr_s, p_csr, q_csr, col_ptr_s, row_idx_s, edge_val_csc_s, p_csc, q_csc, in_feat, W, bias):
    raise NotImplementedError("write your pallas kernel here")


# speedup vs baseline: 1.8383x; 1.8383x over previous
import functools

import jax
import jax.numpy as jnp
from jax import lax
from jax.experimental import pallas as pl
from jax.experimental.pallas import tpu as pltpu
from jax.experimental.pallas import tpu_sc as plsc

_N = 10000
_E = 320000
_D = 128
_NW = 32
_RPW = 320
_RPAD = _NW * _RPW
_K = 128


def _mm_body(x_ref, w_ref, o_ref):
    o_ref[...] = jnp.dot(x_ref[...], w_ref[...],
                         preferred_element_type=jnp.float32)


def _matmul(x, w):
    return pl.pallas_call(
        _mm_body,
        grid=(10,),
        in_specs=[
            pl.BlockSpec((_N // 10, _D), lambda i: (i, 0)),
            pl.BlockSpec((_D, _D), lambda i: (0, 0)),
        ],
        out_specs=pl.BlockSpec((_N // 10, _D), lambda i: (i, 0)),
        out_shape=jax.ShapeDtypeStruct((_N, _D), jnp.float32),
    )(x, w)


def _spmm_body(rp_hbm, ci_hbm, ev_hbm, rid_hbm, h_hbm, b_hbm, out_hbm,
               rpbuf, idxbuf, valbuf, rowbuf, hbuf, acc, bbuf, sem):
    wid = lax.axis_index("s") * 2 + lax.axis_index("c")
    r0 = pl.multiple_of(wid * _RPW, 8)
    pltpu.sync_copy(rp_hbm.at[pl.ds(r0, _RPW + 24)], rpbuf)
    pltpu.sync_copy(b_hbm, bbuf)
    bvecs = [bbuf[pl.ds(d * 16, 16)] for d in range(8)]

    def rp_at(i):
        return rpbuf[pl.ds(i, 16)][0]

    def init_row(rl, c):
        for d in range(8):
            acc[rl, pl.ds(d * 16, 16)] = bvecs[d]
        return c
    lax.fori_loop(0, _RPW, init_row, 0)

    estart = rp_at(0)
    eend = rp_at(_RPW)
    e0 = estart & ~jnp.int32(7)
    nchunks = (eend - e0 + jnp.int32(_K - 1)) // jnp.int32(_K)

    def chunk_body(i, c):
        nat = e0 + i * _K
        base = pl.multiple_of(jnp.minimum(nat, jnp.int32(_E - _K)), 8)
        lo = jnp.maximum(nat, estart)
        hi = jnp.minimum(nat + _K, eend)
        pltpu.sync_copy(ci_hbm.at[pl.ds(base, _K)], idxbuf)
        pltpu.sync_copy(ev_hbm.at[pl.ds(base, _K)], valbuf.at[pl.ds(0, _K)])
        pltpu.sync_copy(rid_hbm.at[pl.ds(base, _K)], rowbuf.at[pl.ds(0, _K)])
        pltpu.async_copy(h_hbm.at[idxbuf], hbuf, sem).wait()

        def edge_body(e, c2):
            li = e - base
            r2 = rowbuf[pl.ds(li, 16)][0] - r0
            vv = jnp.full((16,), valbuf[pl.ds(li, 16)][0], jnp.float32)
            for d in range(8):
                sl = pl.ds(d * 16, 16)
                plsc.addupdate(acc.at[r2, sl], hbuf[li, sl] * vv)
            return c2

        return lax.fori_loop(lo, hi, edge_body, c)

    lax.fori_loop(0, nchunks, chunk_body, 0)

    def relu_row(rl, c):
        for d in range(8):
            sl = pl.ds(d * 16, 16)
            acc[rl, sl] = jnp.maximum(acc[rl, sl], 0.0)
        return c
    lax.fori_loop(0, _RPW, relu_row, 0)
    pltpu.sync_copy(acc, out_hbm.at[pl.ds(r0, _RPW)])


_spmm = functools.partial(
    pl.kernel,
    out_type=jax.ShapeDtypeStruct((_RPAD, _D), jnp.float32),
    mesh=plsc.VectorSubcoreMesh(core_axis_name="c", subcore_axis_name="s"),
    scratch_types=[
        pltpu.VMEM((_RPW + 24,), jnp.int32),
        pltpu.VMEM((_K,), jnp.int32),
        pltpu.VMEM((_K + 16,), jnp.float32),
        pltpu.VMEM((_K + 16,), jnp.int32),
        pltpu.VMEM((_K, _D), jnp.float32),
        pltpu.VMEM((_RPW, _D), jnp.float32),
        pltpu.VMEM((_D,), jnp.float32),
        pltpu.SemaphoreType.DMA,
    ],
)(_spmm_body)


def kernel(nnz, row_ptr_s, col_idx_s, edge_val_csr_s, p_csr, q_csr,
           col_ptr_s, row_idx_s, edge_val_csc_s, p_csc, q_csc,
           in_feat, W, bias):
    rp = row_ptr_s.astype(jnp.int32)
    ci = col_idx_s.astype(jnp.int32)
    ev = edge_val_csr_s.astype(jnp.float32)
    rp_pad = jnp.concatenate(
        [rp, jnp.full((_RPAD + 24 - (_N + 1),), _E, jnp.int32)])
    counts = rp[1:] - rp[:-1]
    rid = jnp.repeat(jnp.arange(_N, dtype=jnp.int32), counts,
                     total_repeat_length=_E)
    h = _matmul(in_feat.astype(jnp.float32), W.astype(jnp.float32))
    out = _spmm(rp_pad, ci, ev, rid, h, bias.astype(jnp.float32))
    return out[:_N]


# speedup vs baseline: 1.9147x; 1.0416x over previous
import functools

import jax
import jax.numpy as jnp
from jax import lax
from jax.experimental import pallas as pl
from jax.experimental.pallas import tpu as pltpu
from jax.experimental.pallas import tpu_sc as plsc

_N = 10000
_E = 320000
_D = 128
_NW = 32
_RPW = 320
_RPAD = _NW * _RPW
_K = 128


def _mm_body(x_ref, w_ref, o_ref):
    o_ref[...] = jnp.dot(x_ref[...], w_ref[...],
                         preferred_element_type=jnp.float32)


def _matmul(x, w):
    return pl.pallas_call(
        _mm_body,
        grid=(10,),
        in_specs=[
            pl.BlockSpec((_N // 10, _D), lambda i: (i, 0)),
            pl.BlockSpec((_D, _D), lambda i: (0, 0)),
        ],
        out_specs=pl.BlockSpec((_N // 10, _D), lambda i: (i, 0)),
        out_shape=jax.ShapeDtypeStruct((_N, _D), jnp.float32),
    )(x, w)


def _spmm_body(rp_hbm, ci_hbm, ev_hbm, rid_hbm, h_hbm, b_hbm, out_hbm,
               rpbuf, idxbuf, valbuf, rowbuf, hbuf, acc, bbuf, sem):
    wid = lax.axis_index("s") * 2 + lax.axis_index("c")
    r0 = pl.multiple_of(wid * _RPW, 8)
    pltpu.sync_copy(rp_hbm.at[pl.ds(r0, _RPW + 24)], rpbuf)
    pltpu.sync_copy(b_hbm, bbuf)
    bvecs = [bbuf[pl.ds(d * 16, 16)] for d in range(8)]

    def rp_at(i):
        return rpbuf[pl.ds(i, 16)][0]

    def init_row(rl, c):
        for d in range(8):
            acc[rl, pl.ds(d * 16, 16)] = bvecs[d]
        return c
    lax.fori_loop(0, _RPW, init_row, 0)

    estart = rp_at(0)
    eend = rp_at(_RPW)
    e0 = estart & ~jnp.int32(7)
    nchunks = (eend - e0 + jnp.int32(_K - 1)) // jnp.int32(_K)

    def chunk_body(i, c):
        nat = e0 + i * _K
        base = pl.multiple_of(jnp.minimum(nat, jnp.int32(_E - _K)), 8)
        lo = jnp.maximum(nat, estart)
        hi = jnp.minimum(nat + _K, eend)
        pltpu.sync_copy(ci_hbm.at[pl.ds(base, _K)], idxbuf)
        pltpu.sync_copy(ev_hbm.at[pl.ds(base, _K)], valbuf)
        pltpu.sync_copy(rid_hbm.at[pl.ds(base, _K)], rowbuf)
        pltpu.async_copy(h_hbm.at[idxbuf], hbuf, sem).wait()

        def grp_body(g, c2):
            li0 = g * 16
            valvec = valbuf[pl.ds(li0, 16)]
            rowvec = rowbuf[pl.ds(li0, 16)]
            for j in range(16):
                e = base + li0 + j
                ok = (e >= lo) & (e < hi)
                r2 = jnp.where(ok, rowvec[j] - r0, jnp.int32(_RPW))
                vv = jnp.full((16,), valvec[j], jnp.float32)
                for d in range(8):
                    sl = pl.ds(d * 16, 16)
                    plsc.addupdate(acc.at[r2, sl], hbuf[li0 + j, sl] * vv)
            return c2

        return lax.fori_loop(0, _K // 16, grp_body, c)

    lax.fori_loop(0, nchunks, chunk_body, 0)

    def relu_row(rl, c):
        for d in range(8):
            sl = pl.ds(d * 16, 16)
            acc[rl, sl] = jnp.maximum(acc[rl, sl], 0.0)
        return c
    lax.fori_loop(0, _RPW, relu_row, 0)
    pltpu.sync_copy(acc.at[pl.ds(0, _RPW)], out_hbm.at[pl.ds(r0, _RPW)])


_spmm = functools.partial(
    pl.kernel,
    out_type=jax.ShapeDtypeStruct((_RPAD, _D), jnp.float32),
    mesh=plsc.VectorSubcoreMesh(core_axis_name="c", subcore_axis_name="s"),
    scratch_types=[
        pltpu.VMEM((_RPW + 24,), jnp.int32),
        pltpu.VMEM((_K,), jnp.int32),
        pltpu.VMEM((_K,), jnp.float32),
        pltpu.VMEM((_K,), jnp.int32),
        pltpu.VMEM((_K, _D), jnp.float32),
        pltpu.VMEM((_RPW + 8, _D), jnp.float32),
        pltpu.VMEM((_D,), jnp.float32),
        pltpu.SemaphoreType.DMA,
    ],
)(_spmm_body)


def kernel(nnz, row_ptr_s, col_idx_s, edge_val_csr_s, p_csr, q_csr,
           col_ptr_s, row_idx_s, edge_val_csc_s, p_csc, q_csc,
           in_feat, W, bias):
    rp = row_ptr_s.astype(jnp.int32)
    ci = col_idx_s.astype(jnp.int32)
    ev = edge_val_csr_s.astype(jnp.float32)
    rp_pad = jnp.concatenate(
        [rp, jnp.full((_RPAD + 24 - (_N + 1),), _E, jnp.int32)])
    counts = rp[1:] - rp[:-1]
    rid = jnp.repeat(jnp.arange(_N, dtype=jnp.int32), counts,
                     total_repeat_length=_E)
    h = _matmul(in_feat.astype(jnp.float32), W.astype(jnp.float32))
    out = _spmm(rp_pad, ci, ev, rid, h, bias.astype(jnp.float32))
    return out[:_N]


# speedup vs baseline: 1.9197x; 1.0026x over previous
import functools

import jax
import jax.numpy as jnp
from jax import lax
from jax.experimental import pallas as pl
from jax.experimental.pallas import tpu as pltpu
from jax.experimental.pallas import tpu_sc as plsc

_N = 10000
_E = 320000
_D = 128
_NW = 32
_RPW = 320
_RPAD = _NW * _RPW
_K = 128


def _mm_body(x_ref, w_ref, o_ref):
    o_ref[...] = jnp.dot(x_ref[...], w_ref[...],
                         preferred_element_type=jnp.float32)


def _matmul(x, w):
    return pl.pallas_call(
        _mm_body,
        grid=(10,),
        in_specs=[
            pl.BlockSpec((_N // 10, _D), lambda i: (i, 0)),
            pl.BlockSpec((_D, _D), lambda i: (0, 0)),
        ],
        out_specs=pl.BlockSpec((_N // 10, _D), lambda i: (i, 0)),
        out_shape=jax.ShapeDtypeStruct((_N, _D), jnp.float32),
    )(x, w)


def _spmm_body(rp_hbm, ci_hbm, ev_hbm, rid_hbm, h_hbm, b_hbm, out_hbm,
               rpbuf, idxbuf, valbuf, rowbuf, hbuf, acc, bbuf, sem):
    wid = lax.axis_index("s") * 2 + lax.axis_index("c")
    r0 = pl.multiple_of(wid * _RPW, 8)
    pltpu.sync_copy(rp_hbm.at[pl.ds(r0, _RPW + 24)], rpbuf)
    pltpu.sync_copy(b_hbm, bbuf)
    bvecs = [bbuf[pl.ds(d * 16, 16)] for d in range(8)]

    def rp_at(i):
        return rpbuf[pl.ds(i, 16)][0]

    def init_row(rl, c):
        for d in range(8):
            acc[rl, pl.ds(d * 16, 16)] = bvecs[d]
        return c
    lax.fori_loop(0, _RPW, init_row, 0)

    def lane_bcast(vec, j):
        dnums = lax.GatherDimensionNumbers(
            offset_dims=(), collapsed_slice_dims=(0,), start_index_map=(0,))
        idx = jnp.full((16, 1), j, jnp.int32)
        return lax.gather(vec, idx, dnums, (1,),
                          mode=lax.GatherScatterMode.PROMISE_IN_BOUNDS)

    iota = lax.iota(jnp.int32, 16)
    cols = [iota + jnp.int32(d * 16) for d in range(8)]
    estart = rp_at(0)
    eend = rp_at(_RPW)
    e0 = estart & ~jnp.int32(7)
    nchunks = (eend - e0 + jnp.int32(_K - 1)) // jnp.int32(_K)

    def chunk_body(i, c):
        nat = e0 + i * _K
        base = pl.multiple_of(jnp.minimum(nat, jnp.int32(_E - _K)), 8)
        lo = jnp.maximum(nat, estart)
        hi = jnp.minimum(nat + _K, eend)
        pltpu.sync_copy(ci_hbm.at[pl.ds(base, _K)], idxbuf)
        pltpu.sync_copy(ev_hbm.at[pl.ds(base, _K)], valbuf)
        pltpu.sync_copy(rid_hbm.at[pl.ds(base, _K)], rowbuf)
        pltpu.async_copy(h_hbm.at[idxbuf], hbuf, sem).wait()

        def grp_body(g, c2):
            li0 = g * 16
            valvec = valbuf[pl.ds(li0, 16)]
            rowvec = rowbuf[pl.ds(li0, 16)] - r0
            for j in range(16):
                vv = lane_bcast(valvec, j)
                rv = lane_bcast(rowvec, j)
                e = base + li0 + j
                ok = (e >= lo) & (e < hi)
                rsel = jnp.where(ok, rv, jnp.int32(_RPW))
                for d in range(8):
                    x = hbuf[li0 + j, pl.ds(d * 16, 16)] * vv
                    plsc.addupdate_scatter(acc, [rsel, cols[d]], x)
            return c2

        return lax.fori_loop(0, _K // 16, grp_body, c)

    lax.fori_loop(0, nchunks, chunk_body, 0)

    def relu_row(rl, c):
        for d in range(8):
            sl = pl.ds(d * 16, 16)
            acc[rl, sl] = jnp.maximum(acc[rl, sl], 0.0)
        return c
    lax.fori_loop(0, _RPW, relu_row, 0)
    pltpu.sync_copy(acc.at[pl.ds(0, _RPW)], out_hbm.at[pl.ds(r0, _RPW)])


_spmm = functools.partial(
    pl.kernel,
    out_type=jax.ShapeDtypeStruct((_RPAD, _D), jnp.float32),
    mesh=plsc.VectorSubcoreMesh(core_axis_name="c", subcore_axis_name="s"),
    compiler_params=pltpu.CompilerParams(needs_layout_passes=False),
    scratch_types=[
        pltpu.VMEM((_RPW + 24,), jnp.int32),
        pltpu.VMEM((_K,), jnp.int32),
        pltpu.VMEM((_K,), jnp.float32),
        pltpu.VMEM((_K,), jnp.int32),
        pltpu.VMEM((_K, _D), jnp.float32),
        pltpu.VMEM((_RPW + 8, _D), jnp.float32),
        pltpu.VMEM((_D,), jnp.float32),
        pltpu.SemaphoreType.DMA,
    ],
)(_spmm_body)


def kernel(nnz, row_ptr_s, col_idx_s, edge_val_csr_s, p_csr, q_csr,
           col_ptr_s, row_idx_s, edge_val_csc_s, p_csc, q_csc,
           in_feat, W, bias):
    rp = row_ptr_s.astype(jnp.int32)
    ci = col_idx_s.astype(jnp.int32)
    ev = edge_val_csr_s.astype(jnp.float32)
    rp_pad = jnp.concatenate(
        [rp, jnp.full((_RPAD + 24 - (_N + 1),), _E, jnp.int32)])
    counts = rp[1:] - rp[:-1]
    rid = jnp.repeat(jnp.arange(_N, dtype=jnp.int32), counts,
                     total_repeat_length=_E)
    h = _matmul(in_feat.astype(jnp.float32), W.astype(jnp.float32))
    out = _spmm(rp_pad, ci, ev, rid, h, bias.astype(jnp.float32))
    return out[:_N]


# speedup vs baseline: 6.4364x; 3.3529x over previous
import functools

import jax
import jax.numpy as jnp
from jax import lax
from jax.experimental import pallas as pl
from jax.experimental.pallas import tpu as pltpu
from jax.experimental.pallas import tpu_sc as plsc

_N = 10000
_E = 320000
_D = 128
_NW = 32
_RPW = 320
_RPAD = _NW * _RPW
_K = 128


def _mm_body(x_ref, w_ref, o_ref):
    o_ref[...] = jnp.dot(x_ref[...], w_ref[...],
                         preferred_element_type=jnp.float32)


def _matmul(x, w):
    return pl.pallas_call(
        _mm_body,
        grid=(10,),
        in_specs=[
            pl.BlockSpec((_N // 10, _D), lambda i: (i, 0)),
            pl.BlockSpec((_D, _D), lambda i: (0, 0)),
        ],
        out_specs=pl.BlockSpec((_N // 10, _D), lambda i: (i, 0)),
        out_shape=jax.ShapeDtypeStruct((_N, _D), jnp.float32),
    )(x, w)


def _spmm_body(rp_hbm, ci_hbm, ev_hbm, rid_hbm, h_hbm, b_hbm, out_hbm,
               rpbuf, idxbuf, valbuf, rowbuf, hbuf, acc, bbuf, sem):
    wid = lax.axis_index("s") * 2 + lax.axis_index("c")
    r0 = pl.multiple_of(wid * _RPW, 8)
    pltpu.sync_copy(rp_hbm.at[pl.ds(r0, _RPW + 24)], rpbuf)
    pltpu.sync_copy(b_hbm, bbuf)
    bvecs = [bbuf[pl.ds(d * 16, 16)] for d in range(8)]

    def rp_at(i):
        return rpbuf[pl.ds(i, 16)][0]

    def init_row(rl, c):
        for d in range(8):
            acc[rl, pl.ds(d * 16, 16)] = bvecs[d]
        return c
    lax.fori_loop(0, _RPW, init_row, 0)

    def lane_bcast(vec, j):
        dnums = lax.GatherDimensionNumbers(
            offset_dims=(), collapsed_slice_dims=(0,), start_index_map=(0,))
        idx = jnp.full((16, 1), j, jnp.int32)
        return lax.gather(vec, idx, dnums, (1,),
                          mode=lax.GatherScatterMode.PROMISE_IN_BOUNDS)

    iota = lax.iota(jnp.int32, 16)
    cols = [iota + jnp.int32(d * 16) for d in range(8)]
    estart = rp_at(0)
    eend = rp_at(_RPW)
    e0 = estart & ~jnp.int32(7)
    nchunks = (eend - e0 + jnp.int32(_K - 1)) // jnp.int32(_K)

    def chunk_body(i, c):
        nat = e0 + i * _K
        base = pl.multiple_of(jnp.minimum(nat, jnp.int32(_E - _K)), 8)
        lo = jnp.maximum(nat, estart)
        hi = jnp.minimum(nat + _K, eend)
        pltpu.sync_copy(ci_hbm.at[pl.ds(base, _K)], idxbuf)
        pltpu.sync_copy(ev_hbm.at[pl.ds(base, _K)], valbuf)
        pltpu.sync_copy(rid_hbm.at[pl.ds(base, _K)], rowbuf)
        pltpu.async_copy(h_hbm.at[idxbuf], hbuf, sem).wait()

        def grp_body(g, c2):
            li0 = g * 16
            valvec = valbuf[pl.ds(li0, 16)]
            rowvec = rowbuf[pl.ds(li0, 16)] - r0
            for j in range(16):
                vv = lane_bcast(valvec, j)
                rv = lane_bcast(rowvec, j)
                e = base + li0 + j
                ok = (e >= lo) & (e < hi)
                rsel = jnp.where(ok, rv, jnp.int32(_RPW))
                for d in range(8):
                    x = hbuf[li0 + j, pl.ds(d * 16, 16)] * vv
                    plsc.addupdate_scatter(acc, [rsel, cols[d]], x)
            return c2

        return lax.fori_loop(0, _K // 16, grp_body, c)

    lax.fori_loop(0, nchunks, chunk_body, 0)

    def relu_row(rl, c):
        for d in range(8):
            sl = pl.ds(d * 16, 16)
            acc[rl, sl] = jnp.maximum(acc[rl, sl], 0.0)
        return c
    lax.fori_loop(0, _RPW, relu_row, 0)
    pltpu.sync_copy(acc.at[pl.ds(0, _RPW)], out_hbm.at[pl.ds(r0, _RPW)])


_spmm = functools.partial(
    pl.kernel,
    out_type=jax.ShapeDtypeStruct((_RPAD, _D), jnp.float32),
    mesh=plsc.VectorSubcoreMesh(core_axis_name="c", subcore_axis_name="s"),
    compiler_params=pltpu.CompilerParams(needs_layout_passes=False),
    scratch_types=[
        pltpu.VMEM((_RPW + 24,), jnp.int32),
        pltpu.VMEM((_K,), jnp.int32),
        pltpu.VMEM((_K,), jnp.float32),
        pltpu.VMEM((_K,), jnp.int32),
        pltpu.VMEM((_K, _D), jnp.float32),
        pltpu.VMEM((_RPW + 8, _D), jnp.float32),
        pltpu.VMEM((_D,), jnp.float32),
        pltpu.SemaphoreType.DMA,
    ],
)(_spmm_body)


def kernel(nnz, row_ptr_s, col_idx_s, edge_val_csr_s, p_csr, q_csr,
           col_ptr_s, row_idx_s, edge_val_csc_s, p_csc, q_csc,
           in_feat, W, bias):
    rp = row_ptr_s.astype(jnp.int32)
    ci = col_idx_s.astype(jnp.int32)
    ev = edge_val_csr_s.astype(jnp.float32)
    rp_pad = jnp.concatenate(
        [rp, jnp.full((_RPAD + 24 - (_N + 1),), _E, jnp.int32)])
    starts = jnp.zeros((_E,), jnp.int32).at[rp[:-1]].max(
        jnp.arange(_N, dtype=jnp.int32), mode='drop')
    rid = lax.cummax(starts, axis=0)
    h = _matmul(in_feat.astype(jnp.float32), W.astype(jnp.float32))
    out = _spmm(rp_pad, ci, ev, rid, h, bias.astype(jnp.float32))
    return out[:_N]


# speedup vs baseline: 8.8130x; 1.3692x over previous
import functools

import jax
import jax.numpy as jnp
from jax import lax
from jax.experimental import pallas as pl
from jax.experimental.pallas import tpu as pltpu
from jax.experimental.pallas import tpu_sc as plsc

_N = 10000
_E = 320000
_D = 128
_NW = 32
_RPW = 320
_RPAD = _NW * _RPW
_K = 128


def _mm_body(x_ref, w_ref, o_ref):
    o_ref[...] = jnp.dot(x_ref[...], w_ref[...],
                         preferred_element_type=jnp.float32)


def _matmul(x, w):
    return pl.pallas_call(
        _mm_body,
        grid=(10,),
        in_specs=[
            pl.BlockSpec((_N // 10, _D), lambda i: (i, 0)),
            pl.BlockSpec((_D, _D), lambda i: (0, 0)),
        ],
        out_specs=pl.BlockSpec((_N // 10, _D), lambda i: (i, 0)),
        out_shape=jax.ShapeDtypeStruct((_N, _D), jnp.float32),
    )(x, w)


def _spmm_body(rp_hbm, ci_hbm, ev_hbm, rid_hbm, h_hbm, b_hbm, out_hbm,
               rpbuf, i0, i1, i2, i3, p0, p1, p2, p3,
               h0, h1, h2, h3, acc, bbuf,
               g0, g1, g2, g3, s0, s1, s2, s3):
    idxb = [i0, i1, i2, i3]
    pkb = [p0, p1, p2, p3]
    hb = [h0, h1, h2, h3]
    gsem = [g0, g1, g2, g3]
    ssem = [s0, s1, s2, s3]

    wid = lax.axis_index("s") * 2 + lax.axis_index("c")
    r0 = pl.multiple_of(wid * _RPW, 8)
    pltpu.sync_copy(rp_hbm.at[pl.ds(r0, _RPW + 24)], rpbuf)
    pltpu.sync_copy(b_hbm, bbuf)
    bvecs = [bbuf[pl.ds(d * 16, 16)] for d in range(8)]

    def rp_at(i):
        return rpbuf[pl.ds(i, 16)][0]

    def init_row(rl, c):
        for d in range(8):
            acc[rl, pl.ds(d * 16, 16)] = bvecs[d]
        return c
    lax.fori_loop(0, _RPW, init_row, 0)

    def lane_bcast(vec, j):
        dnums = lax.GatherDimensionNumbers(
            offset_dims=(), collapsed_slice_dims=(0,), start_index_map=(0,))
        idx = jnp.full((16, 1), j, jnp.int32)
        return lax.gather(vec, idx, dnums, (1,),
                          mode=lax.GatherScatterMode.PROMISE_IN_BOUNDS)

    iota = lax.iota(jnp.int32, 16)
    cols = [iota + jnp.int32(d * 16) for d in range(8)]
    estart = rp_at(0)
    eend = rp_at(_RPW)
    e0 = estart & ~jnp.int32(7)
    nchunks = (eend - e0 + jnp.int32(_K - 1)) // jnp.int32(_K)

    def cbase(c):
        nat = e0 + c * _K
        base = pl.multiple_of(jnp.minimum(nat, jnp.int32(_E - _K)), 8)
        return base, nat

    def stage(c, b):
        base, _ = cbase(c)
        pltpu.async_copy(ci_hbm.at[pl.ds(base, _K)], idxb[b], ssem[b])
        pltpu.async_copy(ev_hbm.at[pl.ds(base, _K)], pkb[b].at[0], ssem[b])
        pltpu.async_copy(rid_hbm.at[pl.ds(base, _K)], pkb[b].at[1], ssem[b])

    def wait_stage(b):
        pltpu.make_async_copy(ci_hbm.at[pl.ds(0, _K)], idxb[b], ssem[b]).wait()
        pltpu.make_async_copy(ev_hbm.at[pl.ds(0, _K)], pkb[b].at[0],
                              ssem[b]).wait()
        pltpu.make_async_copy(rid_hbm.at[pl.ds(0, _K)], pkb[b].at[1],
                              ssem[b]).wait()

    def gather(b):
        pltpu.async_copy(h_hbm.at[idxb[b]], hb[b], gsem[b])

    def wait_gather(b):
        pltpu.make_async_copy(h_hbm.at[idxb[b]], hb[b], gsem[b]).wait()

    def compute(c, b):
        base, nat = cbase(c)
        lo = jnp.maximum(nat, estart)
        hi = jnp.minimum(nat + _K, eend)
        hbuf = hb[b]
        pkbuf = pkb[b]

        def grp_body(g, c2):
            li0 = g * 16
            valvec = plsc.bitcast(pkbuf[0, pl.ds(li0, 16)], jnp.float32)
            rowvec = pkbuf[1, pl.ds(li0, 16)] - r0
            for j in range(16):
                vv = lane_bcast(valvec, j)
                rv = lane_bcast(rowvec, j)
                e = base + li0 + j
                ok = (e >= lo) & (e < hi)
                rsel = jnp.where(ok, rv, jnp.int32(_RPW))
                for d in range(8):
                    x = hbuf[li0 + j, pl.ds(d * 16, 16)] * vv
                    plsc.addupdate_scatter(acc, [rsel, cols[d]], x)
            return c2

        lax.fori_loop(0, _K // 16, grp_body, 0)

    for b in range(4):
        stage(b, b)
    wait_stage(0)
    gather(0)

    niter4 = (nchunks + jnp.int32(3)) // jnp.int32(4)

    def quad_body(q, c):
        for b in range(4):
            cid = q * 4 + b
            nb = (b + 1) % 4
            wait_stage(nb)
            gather(nb)
            wait_gather(b)
            compute(cid, b)
            stage(cid + 4, b)
        return c

    lax.fori_loop(0, niter4, quad_body, 0)

    for b in (1, 2, 3):
        wait_stage(b)
    wait_gather(0)

    def relu_row(rl, c):
        for d in range(8):
            sl = pl.ds(d * 16, 16)
            acc[rl, sl] = jnp.maximum(acc[rl, sl], 0.0)
        return c
    lax.fori_loop(0, _RPW, relu_row, 0)
    pltpu.sync_copy(acc.at[pl.ds(0, _RPW)], out_hbm.at[pl.ds(r0, _RPW)])


_spmm = functools.partial(
    pl.kernel,
    out_type=jax.ShapeDtypeStruct((_RPAD, _D), jnp.float32),
    mesh=plsc.VectorSubcoreMesh(core_axis_name="c", subcore_axis_name="s"),
    compiler_params=pltpu.CompilerParams(needs_layout_passes=False),
    scratch_types=(
        [pltpu.VMEM((_RPW + 24,), jnp.int32)]
        + [pltpu.VMEM((_K,), jnp.int32)] * 4
        + [pltpu.VMEM((2, _K), jnp.int32)] * 4
        + [pltpu.VMEM((_K, _D), jnp.float32)] * 4
        + [pltpu.VMEM((_RPW + 8, _D), jnp.float32)]
        + [pltpu.VMEM((_D,), jnp.float32)]
        + [pltpu.SemaphoreType.DMA] * 8
    ),
)(_spmm_body)


def kernel(nnz, row_ptr_s, col_idx_s, edge_val_csr_s, p_csr, q_csr,
           col_ptr_s, row_idx_s, edge_val_csc_s, p_csc, q_csc,
           in_feat, W, bias):
    rp = row_ptr_s.astype(jnp.int32)
    ci = col_idx_s.astype(jnp.int32)
    ev = edge_val_csr_s.astype(jnp.float32)
    rp_pad = jnp.concatenate(
        [rp, jnp.full((_RPAD + 24 - (_N + 1),), _E, jnp.int32)])
    starts = jnp.zeros((_E,), jnp.int32).at[rp[:-1]].max(
        jnp.arange(_N, dtype=jnp.int32), mode='drop')
    rid = lax.cummax(starts, axis=0)
    evi = lax.bitcast_convert_type(ev, jnp.int32)
    h = _matmul(in_feat.astype(jnp.float32), W.astype(jnp.float32))
    out = _spmm(rp_pad, ci, evi, rid, h, bias.astype(jnp.float32))
    return out[:_N]


# speedup vs baseline: 19.9583x; 2.2646x over previous
import functools

import jax
import jax.numpy as jnp
from jax import lax
from jax.experimental import pallas as pl
from jax.experimental.pallas import tpu as pltpu
from jax.experimental.pallas import tpu_sc as plsc

_N = 10000
_E = 320000
_D = 128
_NW = 32
_RPW = 320
_RPAD = _NW * _RPW
_K = 128


def _mm_body(x_ref, w_ref, o_ref):
    o_ref[...] = jnp.dot(x_ref[...], w_ref[...],
                         preferred_element_type=jnp.float32)


def _matmul(x, w):
    return pl.pallas_call(
        _mm_body,
        grid=(10,),
        in_specs=[
            pl.BlockSpec((_N // 10, _D), lambda i: (i, 0)),
            pl.BlockSpec((_D, _D), lambda i: (0, 0)),
        ],
        out_specs=pl.BlockSpec((_N // 10, _D), lambda i: (i, 0)),
        out_shape=jax.ShapeDtypeStruct((_N, _D), jnp.float32),
    )(x, w)


def _spmm_body(rp_hbm, ci_hbm, ev_hbm, rid_hbm, h_hbm, b_hbm, out_hbm,
               rpbuf, i0, i1, i2, i3, p0, p1, p2, p3,
               h0, h1, h2, h3, r0b, r1b, r2b, r3b, acc_sh, rbuf, bbuf,
               g0, g1, g2, g3, s0, s1, s2, s3, c0, c1, c2, c3):
    idxb = [i0, i1, i2, i3]
    pkb = [p0, p1, p2, p3]
    hb = [h0, h1, h2, h3]
    rloc = [r0b, r1b, r2b, r3b]
    gsem = [g0, g1, g2, g3]
    ssem = [s0, s1, s2, s3]
    csem = [c0, c1, c2, c3]

    sid = lax.axis_index("s")
    wid = sid * 2 + lax.axis_index("c")
    r0 = pl.multiple_of(wid * _RPW, 8)
    aoff = pl.multiple_of(sid * (_RPW + 8), 8)
    pltpu.sync_copy(rp_hbm.at[pl.ds(r0, _RPW + 24)], rpbuf)
    pltpu.sync_copy(b_hbm, bbuf)
    bvecs = [bbuf[pl.ds(d * 16, 16)] for d in range(8)]

    def rp_at(i):
        return rpbuf[pl.ds(i, 16)][0]

    zv = jnp.zeros((16,), jnp.float32)

    def zero_rbuf(rl, c):
        for d in range(8):
            rbuf[rl, pl.ds(d * 16, 16)] = zv
        return c
    lax.fori_loop(0, 64, zero_rbuf, 0)
    for blk in range(5):
        pltpu.sync_copy(rbuf, acc_sh.at[pl.ds(aoff + blk * 64, 64)])
    pltpu.sync_copy(rbuf.at[pl.ds(0, 8)],
                    acc_sh.at[pl.ds(aoff + 320, 8)])

    def lane_bcast(vec, j):
        dnums = lax.GatherDimensionNumbers(
            offset_dims=(), collapsed_slice_dims=(0,), start_index_map=(0,))
        idx = jnp.full((16, 1), j, jnp.int32)
        return lax.gather(vec, idx, dnums, (1,),
                          mode=lax.GatherScatterMode.PROMISE_IN_BOUNDS)

    iota = lax.iota(jnp.int32, 16)
    cols = [iota + jnp.int32(d * 16) for d in range(8)]
    estart = rp_at(0)
    eend = rp_at(_RPW)
    e0 = estart & ~jnp.int32(7)
    nchunks = (eend - e0 + jnp.int32(_K - 1)) // jnp.int32(_K)

    def cbase(c):
        nat = e0 + c * _K
        base = pl.multiple_of(jnp.minimum(nat, jnp.int32(_E - _K)), 8)
        return base, nat

    def stage(c, b):
        base, _ = cbase(c)
        pltpu.async_copy(ci_hbm.at[pl.ds(base, _K)], idxb[b], ssem[b])
        pltpu.async_copy(ev_hbm.at[pl.ds(base, _K)], pkb[b].at[0], ssem[b])
        pltpu.async_copy(rid_hbm.at[pl.ds(base, _K)], pkb[b].at[1], ssem[b])

    def wait_stage(b):
        pltpu.make_async_copy(ci_hbm.at[pl.ds(0, _K)], idxb[b], ssem[b]).wait()
        pltpu.make_async_copy(ev_hbm.at[pl.ds(0, _K)], pkb[b].at[0],
                              ssem[b]).wait()
        pltpu.make_async_copy(rid_hbm.at[pl.ds(0, _K)], pkb[b].at[1],
                              ssem[b]).wait()

    def gather(b):
        pltpu.async_copy(h_hbm.at[idxb[b]], hb[b], gsem[b])

    def wait_gather(b):
        pltpu.make_async_copy(h_hbm.at[idxb[b]], hb[b], gsem[b]).wait()

    def scat(b):
        pltpu.async_copy(hb[b], acc_sh.at[rloc[b]], csem[b], add=True)

    def wait_scat(b):
        pltpu.make_async_copy(hb[b], acc_sh.at[rloc[b]], csem[b]).wait()

    def compute(c, b):
        base, nat = cbase(c)
        lo = jnp.maximum(nat, estart)
        hi = jnp.minimum(nat + _K, eend)
        hbuf = hb[b]
        pkbuf = pkb[b]
        rlb = rloc[b]

        def grp_body(g, c2):
            li0 = g * 16
            valvec = plsc.bitcast(pkbuf[0, pl.ds(li0, 16)], jnp.float32)
            rowvec = pkbuf[1, pl.ds(li0, 16)] - r0
            ev16 = jnp.full((16,), base + li0, jnp.int32) + iota
            okv = (ev16 >= lo) & (ev16 < hi)
            rlb[pl.ds(li0, 16)] = aoff + jnp.where(okv, rowvec,
                                                   jnp.int32(_RPW))
            for j in range(16):
                vv = lane_bcast(valvec, j)
                for d in range(8):
                    sl = pl.ds(d * 16, 16)
                    hbuf[li0 + j, sl] = hbuf[li0 + j, sl] * vv
            return c2

        lax.fori_loop(0, _K // 16, grp_body, 0)

    dump = jnp.full((16,), _RPW, jnp.int32) + aoff
    for b in range(4):
        for g in range(_K // 16):
            rloc[b][pl.ds(g * 16, 16)] = dump
    for b in range(4):
        stage(b, b)
        scat(b)
    wait_stage(0)
    gather(0)

    niter4 = (nchunks + jnp.int32(3)) // jnp.int32(4)

    def quad_body(q, c):
        for b in range(4):
            cid = q * 4 + b
            nb = (b + 1) % 4
            wait_stage(nb)
            wait_scat(nb)
            gather(nb)
            wait_gather(b)
            compute(cid, b)
            scat(b)
            stage(cid + 4, b)
        return c

    lax.fori_loop(0, niter4, quad_body, 0)

    for b in (1, 2, 3):
        wait_stage(b)
    wait_gather(0)
    for b in range(4):
        wait_scat(b)

    for blk in range(5):
        pltpu.sync_copy(acc_sh.at[pl.ds(aoff + blk * 64, 64)], rbuf)

        def finish_row(rl, c):
            for d in range(8):
                sl = pl.ds(d * 16, 16)
                rbuf[rl, sl] = jnp.maximum(rbuf[rl, sl] + bvecs[d], 0.0)
            return c
        lax.fori_loop(0, 64, finish_row, 0)
        pltpu.sync_copy(rbuf, out_hbm.at[pl.ds(r0 + blk * 64, 64)])


_spmm = functools.partial(
    pl.kernel,
    out_type=jax.ShapeDtypeStruct((_RPAD, _D), jnp.float32),
    mesh=plsc.VectorSubcoreMesh(core_axis_name="c", subcore_axis_name="s"),
    compiler_params=pltpu.CompilerParams(needs_layout_passes=False),
    scratch_types=(
        [pltpu.VMEM((_RPW + 24,), jnp.int32)]
        + [pltpu.VMEM((_K,), jnp.int32)] * 4
        + [pltpu.VMEM((2, _K), jnp.int32)] * 4
        + [pltpu.VMEM((_K, _D), jnp.float32)] * 4
        + [pltpu.VMEM((_K,), jnp.int32)] * 4
        + [pltpu.VMEM_SHARED((16 * (_RPW + 8), _D), jnp.float32)]
        + [pltpu.VMEM((64, _D), jnp.float32)]
        + [pltpu.VMEM((_D,), jnp.float32)]
        + [pltpu.SemaphoreType.DMA] * 12
    ),
)(_spmm_body)


def kernel(nnz, row_ptr_s, col_idx_s, edge_val_csr_s, p_csr, q_csr,
           col_ptr_s, row_idx_s, edge_val_csc_s, p_csc, q_csc,
           in_feat, W, bias):
    rp = row_ptr_s.astype(jnp.int32)
    ci = col_idx_s.astype(jnp.int32)
    ev = edge_val_csr_s.astype(jnp.float32)
    rp_pad = jnp.concatenate(
        [rp, jnp.full((_RPAD + 24 - (_N + 1),), _E, jnp.int32)])
    starts = jnp.zeros((_E,), jnp.int32).at[rp[:-1]].max(
        jnp.arange(_N, dtype=jnp.int32), mode='drop')
    rid = lax.cummax(starts, axis=0)
    evi = lax.bitcast_convert_type(ev, jnp.int32)
    h = _matmul(in_feat.astype(jnp.float32), W.astype(jnp.float32))
    out = _spmm(rp_pad, ci, evi, rid, h, bias.astype(jnp.float32))
    return out[:_N]


# speedup vs baseline: 23.1264x; 1.1587x over previous
import functools

import jax
import jax.numpy as jnp
from jax import lax
from jax.experimental import pallas as pl
from jax.experimental.pallas import tpu as pltpu
from jax.experimental.pallas import tpu_sc as plsc

_N = 10000
_E = 320000
_D = 128
_NW = 32
_RPW = 320
_RPAD = _NW * _RPW
_K = 128


def _mm_body(x_ref, w_ref, o_ref):
    o_ref[...] = jnp.dot(x_ref[...], w_ref[...],
                         preferred_element_type=jnp.float32)


def _matmul(x, w):
    return pl.pallas_call(
        _mm_body,
        grid=(10,),
        in_specs=[
            pl.BlockSpec((_N // 10, _D), lambda i: (i, 0)),
            pl.BlockSpec((_D, _D), lambda i: (0, 0)),
        ],
        out_specs=pl.BlockSpec((_N // 10, _D), lambda i: (i, 0)),
        out_shape=jax.ShapeDtypeStruct((_N, _D), jnp.float32),
    )(x, w)


def _spmm_body(rp_hbm, ci_hbm, ev_hbm, rid_hbm, h_hbm, b_hbm, out_hbm,
               rpbuf, i0, i1, i2, i3, p0, p1, p2, p3,
               h0, h1, h2, h3, r0b, r1b, r2b, r3b, acc_sh, rbuf, bbuf,
               g0, g1, g2, g3, s0, s1, s2, s3, c0, c1, c2, c3):
    idxb = [i0, i1, i2, i3]
    pkb = [p0, p1, p2, p3]
    hb = [h0, h1, h2, h3]
    rloc = [r0b, r1b, r2b, r3b]
    gsem = [g0, g1, g2, g3]
    ssem = [s0, s1, s2, s3]
    csem = [c0, c1, c2, c3]

    sid = lax.axis_index("s")
    wid = sid * 2 + lax.axis_index("c")
    r0 = pl.multiple_of(wid * _RPW, 8)
    aoff = pl.multiple_of(sid * (_RPW + 8), 8)
    pltpu.sync_copy(rp_hbm.at[pl.ds(r0, _RPW + 24)], rpbuf)
    pltpu.sync_copy(b_hbm, bbuf)
    bvecs = [bbuf[pl.ds(d * 16, 16)] for d in range(8)]

    def rp_at(i):
        return rpbuf[pl.ds(i, 16)][0]

    zv = jnp.zeros((16,), jnp.float32)

    def zero_rbuf(rl, c):
        for d in range(8):
            rbuf[rl, pl.ds(d * 16, 16)] = zv
        return c
    lax.fori_loop(0, 64, zero_rbuf, 0)
    for blk in range(5):
        pltpu.sync_copy(rbuf, acc_sh.at[pl.ds(aoff + blk * 64, 64)])
    pltpu.sync_copy(rbuf.at[pl.ds(0, 8)],
                    acc_sh.at[pl.ds(aoff + 320, 8)])

    def lane_bcast(vec, j):
        dnums = lax.GatherDimensionNumbers(
            offset_dims=(), collapsed_slice_dims=(0,), start_index_map=(0,))
        idx = jnp.full((16, 1), j, jnp.int32)
        return lax.gather(vec, idx, dnums, (1,),
                          mode=lax.GatherScatterMode.PROMISE_IN_BOUNDS)

    iota = lax.iota(jnp.int32, 16)
    cols = [iota + jnp.int32(d * 16) for d in range(8)]
    estart = rp_at(0)
    eend = rp_at(_RPW)
    e0 = estart & ~jnp.int32(7)
    nchunks = (eend - e0 + jnp.int32(_K - 1)) // jnp.int32(_K)

    def cbase(c):
        nat = e0 + c * _K
        base = pl.multiple_of(jnp.minimum(nat, jnp.int32(_E - _K)), 8)
        return base, nat

    def stage(c, b):
        base, _ = cbase(c)
        pltpu.async_copy(ci_hbm.at[pl.ds(base, _K)], idxb[b], ssem[b])
        pltpu.async_copy(ev_hbm.at[pl.ds(base, _K)], pkb[b].at[0], ssem[b])
        pltpu.async_copy(rid_hbm.at[pl.ds(base, _K)], pkb[b].at[1], ssem[b])

    def wait_stage(b):
        pltpu.make_async_copy(ci_hbm.at[pl.ds(0, _K)], idxb[b], ssem[b]).wait()
        pltpu.make_async_copy(ev_hbm.at[pl.ds(0, _K)], pkb[b].at[0],
                              ssem[b]).wait()
        pltpu.make_async_copy(rid_hbm.at[pl.ds(0, _K)], pkb[b].at[1],
                              ssem[b]).wait()

    def gather(b):
        pltpu.async_copy(h_hbm.at[idxb[b]], hb[b], gsem[b])

    def wait_gather(b):
        pltpu.make_async_copy(h_hbm.at[idxb[b]], hb[b], gsem[b]).wait()

    def scat(b):
        pltpu.async_copy(hb[b], acc_sh.at[rloc[b]], csem[b], add=True)

    def wait_scat(b):
        pltpu.make_async_copy(hb[b], acc_sh.at[rloc[b]], csem[b]).wait()

    def compute(c, b):
        base, nat = cbase(c)
        lo = jnp.maximum(nat, estart)
        hi = jnp.minimum(nat + _K, eend)
        hbuf = hb[b]
        pkbuf = pkb[b]
        rlb = rloc[b]

        def grp_body(g, c2):
            li0 = g * 16
            valvec = plsc.bitcast(pkbuf[0, pl.ds(li0, 16)], jnp.float32)
            rowvec = pkbuf[1, pl.ds(li0, 16)] - r0
            ev16 = jnp.full((16,), base + li0, jnp.int32) + iota
            okv = (ev16 >= lo) & (ev16 < hi)
            rlb[pl.ds(li0, 16)] = aoff + jnp.where(okv, rowvec,
                                                   jnp.int32(_RPW))
            for j in range(16):
                vv = lane_bcast(valvec, j)
                for d in range(8):
                    sl = pl.ds(d * 16, 16)
                    hbuf[li0 + j, sl] = hbuf[li0 + j, sl] * vv
            return c2

        lax.fori_loop(0, _K // 16, grp_body, 0)

    dump = jnp.full((16,), _RPW, jnp.int32) + aoff
    for b in range(4):
        for g in range(_K // 16):
            rloc[b][pl.ds(g * 16, 16)] = dump
    for b in range(4):
        stage(b, b)
        scat(b)
    wait_stage(0)
    gather(0)

    niter4 = (nchunks + jnp.int32(3)) // jnp.int32(4)

    def quad_body(q, c):
        for b in range(4):
            cid = q * 4 + b
            nb = (b + 1) % 4
            wait_stage(nb)
            wait_scat(nb)
            gather(nb)
            wait_gather(b)
            compute(cid, b)
            scat(b)
            stage(cid + 4, b)
        return c

    lax.fori_loop(0, niter4, quad_body, 0)

    for b in (1, 2, 3):
        wait_stage(b)
    wait_gather(0)
    for b in range(4):
        wait_scat(b)

    for blk in range(5):
        pltpu.sync_copy(acc_sh.at[pl.ds(aoff + blk * 64, 64)], rbuf)

        def finish_row(rl, c):
            for d in range(8):
                sl = pl.ds(d * 16, 16)
                rbuf[rl, sl] = jnp.maximum(rbuf[rl, sl] + bvecs[d], 0.0)
            return c
        lax.fori_loop(0, 64, finish_row, 0)
        pltpu.sync_copy(rbuf, out_hbm.at[pl.ds(r0 + blk * 64, 64)])


_spmm = functools.partial(
    pl.kernel,
    out_type=jax.ShapeDtypeStruct((_RPAD, _D), jnp.float32),
    mesh=plsc.VectorSubcoreMesh(core_axis_name="c", subcore_axis_name="s"),
    compiler_params=pltpu.CompilerParams(needs_layout_passes=False),
    scratch_types=(
        [pltpu.VMEM((_RPW + 24,), jnp.int32)]
        + [pltpu.VMEM((_K,), jnp.int32)] * 4
        + [pltpu.VMEM((2, _K), jnp.int32)] * 4
        + [pltpu.VMEM((_K, _D), jnp.float32)] * 4
        + [pltpu.VMEM((_K,), jnp.int32)] * 4
        + [pltpu.VMEM_SHARED((16 * (_RPW + 8), _D), jnp.float32)]
        + [pltpu.VMEM((64, _D), jnp.float32)]
        + [pltpu.VMEM((_D,), jnp.float32)]
        + [pltpu.SemaphoreType.DMA] * 12
    ),
)(_spmm_body)


def kernel(nnz, row_ptr_s, col_idx_s, edge_val_csr_s, p_csr, q_csr,
           col_ptr_s, row_idx_s, edge_val_csc_s, p_csc, q_csc,
           in_feat, W, bias):
    rp = row_ptr_s.astype(jnp.int32)
    ci = col_idx_s.astype(jnp.int32)
    ev = edge_val_csr_s.astype(jnp.float32)
    rp_pad = jnp.concatenate(
        [rp, jnp.full((_RPAD + 24 - (_N + 1),), _E, jnp.int32)])
    pos = jnp.where(rp[1:] > rp[:-1], rp[:-1], _E)
    starts = jnp.zeros((_E,), jnp.int32).at[pos].max(
        jnp.arange(_N, dtype=jnp.int32), mode='drop', unique_indices=True)
    rid = lax.cummax(starts, axis=0)
    evi = lax.bitcast_convert_type(ev, jnp.int32)
    h = _matmul(in_feat.astype(jnp.float32), W.astype(jnp.float32))
    out = _spmm(rp_pad, ci, evi, rid, h, bias.astype(jnp.float32))
    return out[:_N]
